# baseline (speedup 1.0000x reference)
import functools

import jax
import jax.numpy as jnp
from jax import lax
from jax.experimental import pallas as pl
from jax.experimental.pallas import tpu as pltpu
from jax.experimental.pallas import tpu_sc as plsc

_N = 10000
_D = 128
_E = 320000
_P = 100000

_NC, _NS = 2, 16
_NW = _NC * _NS

_ACC = 10240
_STRIPE = _ACC // _NS
_DUMMY = _ACC - 1

_CHUNK = 128
_EROWS = 2560
_EPT = _EROWS // _NW
_EPAD = _EROWS * _CHUNK

_PROWS = 896
_PPAD = _PROWS * _CHUNK
_PRT = 2 * _PROWS // _NW

_B = 1000
_DH = 64

_mesh = plsc.VectorSubcoreMesh(core_axis_name="c", subcore_axis_name="s")


def _sc_hist(s_idx, r_idx, ones16, z16):
    @functools.partial(
        pl.kernel,
        out_type=jax.ShapeDtypeStruct((_NC, 2, _ACC, 16), jnp.float32),
        mesh=_mesh,
        compiler_params=pltpu.CompilerParams(use_tc_tiling_on_sc=False),
        scratch_types=[
            pltpu.VMEM((_EPT, _CHUNK), jnp.int32),
            pltpu.VMEM((_EPT, _CHUNK), jnp.int32),
            pltpu.VMEM((_CHUNK, 16), jnp.float32),
            pltpu.VMEM_SHARED((_ACC, 16), jnp.float32),
            pltpu.VMEM_SHARED((_ACC, 16), jnp.float32),
        ] + [pltpu.SemaphoreType.DMA] * 8,
    )
    def hist_kernel(s_hbm, r_hbm, ones_hbm, z_hbm, out_hbm, s_v, r_v, ones_v,
                    acc_s, acc_r, *hsems):
        cid = lax.axis_index("c")
        sid = lax.axis_index("s")
        row0 = (sid * _NC + cid) * _EPT
        stripe = pl.ds(sid * _STRIPE, _STRIPE)
        pltpu.sync_copy(z_hbm, acc_s.at[stripe])
        pltpu.sync_copy(z_hbm, acc_r.at[stripe])
        pltpu.sync_copy(s_hbm.at[pl.ds(row0, _EPT)], s_v)
        pltpu.sync_copy(r_hbm.at[pl.ds(row0, _EPT)], r_v)
        pltpu.sync_copy(ones_hbm, ones_v)
        plsc.subcore_barrier()

        @pl.loop(0, _EPT, step=4)
        def _(j):
            cps = []
            for k in range(4):
                cps.append(pltpu.async_copy(
                    ones_v, acc_s.at[s_v.at[j + k]], hsems[k], add=True))
                cps.append(pltpu.async_copy(
                    ones_v, acc_r.at[r_v.at[j + k]], hsems[4 + k], add=True))
            for cp in cps:
                cp.wait()

        plsc.subcore_barrier()
        pltpu.sync_copy(acc_s.at[stripe], out_hbm.at[cid, 0, stripe])
        pltpu.sync_copy(acc_r.at[stripe], out_hbm.at[cid, 1, stripe])

    return hist_kernel(s_idx, r_idx, ones16, z16)


def _sc_scatter(src, s_idx, r_idx, z64):
    @functools.partial(
        pl.kernel,
        out_type=jax.ShapeDtypeStruct((_NC, _ACC, _DH), jnp.float32),
        mesh=_mesh,
        compiler_params=pltpu.CompilerParams(use_tc_tiling_on_sc=False),
        scratch_types=[
            pltpu.VMEM((_EPT, _CHUNK), jnp.int32),
            pltpu.VMEM((_EPT, _CHUNK), jnp.int32),
        ] + [pltpu.VMEM((_CHUNK, _DH), jnp.float32)] * 8
          + [pltpu.VMEM_SHARED((_ACC, _DH), jnp.float32)]
          + [pltpu.SemaphoreType.DMA] * 16,
    )
    def scat_kernel(x_hbm, s_hbm, r_hbm, z_hbm, out_hbm, s_v, r_v, *rest):
        bufs = rest[:8]
        acc = rest[8]
        gsems = rest[9:17]
        ssems = rest[17:25]
        cid = lax.axis_index("c")
        sid = lax.axis_index("s")
        row0 = (sid * _NC + cid) * _EPT
        stripe = pl.ds(sid * _STRIPE, _STRIPE)
        pltpu.sync_copy(z_hbm, acc.at[stripe])
        pltpu.sync_copy(s_hbm.at[pl.ds(row0, _EPT)], s_v)
        pltpu.sync_copy(r_hbm.at[pl.ds(row0, _EPT)], r_v)
        plsc.subcore_barrier()

        @pl.loop(0, _EPT, step=8)
        def _(j):
            gs = [
                pltpu.async_copy(x_hbm.at[s_v.at[j + k]], bufs[k], gsems[k])
                for k in range(8)
            ]
            ss = []
            for k in range(8):
                gs[k].wait()
                ss.append(pltpu.async_copy(
                    bufs[k], acc.at[r_v.at[j + k]], ssems[k], add=True))
            for cp in ss:
                cp.wait()

        plsc.subcore_barrier()
        pltpu.sync_copy(acc.at[stripe], out_hbm.at[cid, stripe])

    return scat_kernel(src, s_idx, r_idx, z64)


def _sc_pair_gather(h_lo, h_hi, p_idx):
    o = jax.ShapeDtypeStruct((2 * _PPAD, _DH), jnp.float32)

    @functools.partial(
        pl.kernel,
        out_type=[o, o],
        mesh=_mesh,
        compiler_params=pltpu.CompilerParams(use_tc_tiling_on_sc=False),
        scratch_types=[
            pltpu.VMEM((_PRT, _CHUNK), jnp.int32),
        ] + [pltpu.VMEM((_CHUNK, _DH), jnp.float32)] * 8
          + [pltpu.SemaphoreType.DMA] * 16,
    )
    def pg_kernel(lo_hbm, hi_hbm, i_hbm, out_lo, out_hi, i_v, *rest):
        bufs = rest[:8]
        gsems = rest[8:16]
        wsems = rest[16:24]
        cid = lax.axis_index("c")
        sid = lax.axis_index("s")
        row0 = (sid * _NC + cid) * _PRT
        pltpu.sync_copy(i_hbm.at[pl.ds(row0, _PRT)], i_v)
        srcs = (lo_hbm, hi_hbm)
        outs = (out_lo, out_hi)

        @pl.loop(0, _PRT, step=4)
        def _(j):
            gs = [
                pltpu.async_copy(srcs[k % 2].at[i_v.at[j + k // 2]],
                                 bufs[k], gsems[k])
                for k in range(8)
            ]
            ws = []
            for k in range(8):
                gs[k].wait()
                ws.append(pltpu.async_copy(
                    bufs[k],
                    outs[k % 2].at[pl.ds((row0 + j + k // 2) * _CHUNK, _CHUNK)],
                    wsems[k]))
            for cp in ws:
                cp.wait()

    return pg_kernel(h_lo, h_hi, p_idx)


def _tc_prep(hist, emb):
    def body(hs_ref, hr_ref, emb_ref, xn_ref, ci_ref, rs_ref):
        hsb = hs_ref[...]
        hrb = hr_ref[...]
        deg = hsb[0, 0, :, 0:1] + hsb[1, 0, :, 0:1] + 1.0
        cnt = hrb[0, 0, :, 0:1] + hrb[1, 0, :, 0:1] + 1.0
        rs = jnp.broadcast_to(lax.rsqrt(deg), (_B, _D))
        t = jnp.broadcast_to(lax.rsqrt(cnt), (_B, _D))
        xn_ref[...] = emb_ref[...] * rs
        ci_ref[...] = t * t * t
        rs_ref[...] = rs

    o = jax.ShapeDtypeStruct((_N, _D), jnp.float32)
    return pl.pallas_call(
        body,
        grid=(_N // _B,),
        in_specs=[
            pl.BlockSpec((_NC, 1, _B, 16), lambda i: (0, 0, i, 0)),
            pl.BlockSpec((_NC, 1, _B, 16), lambda i: (0, 1, i, 0)),
            pl.BlockSpec((_B, _D), lambda i: (i, 0)),
        ],
        out_specs=[pl.BlockSpec((_B, _D), lambda i: (i, 0))] * 3,
        out_shape=[o, o, o],
    )(hist, hist, emb)


def _tc_layer1(x, parts_a, parts_b, xn, ci, rs, WaT, WbT, b):
    def body(x_ref, pa_ref, pb_ref, xn_ref, ci_ref, rs_ref, wa_ref, wb_ref,
             b_ref, h_ref, xn2_ref):
        pa = pa_ref[...]
        pb = pb_ref[...]
        summed = jnp.concatenate([pa[0] + pa[1], pb[0] + pb[1]], axis=-1)
        xu = (summed + xn_ref[...]) * ci_ref[...]
        h = jnp.dot(x_ref[...], wa_ref[...], preferred_element_type=jnp.float32,
                    precision=lax.Precision.HIGHEST)
        h = h + jnp.dot(xu, wb_ref[...], preferred_element_type=jnp.float32,
                        precision=lax.Precision.HIGHEST)
        h = jnp.maximum(h + b_ref[...], 0.0)
        h_ref[...] = h
        xn2_ref[...] = h * rs_ref[...]

    o = jax.ShapeDtypeStruct((_N, _D), jnp.float32)
    return pl.pallas_call(
        body,
        grid=(_N // _B,),
        in_specs=[
            pl.BlockSpec((_B, _D), lambda i: (i, 0)),
            pl.BlockSpec((_NC, _B, _DH), lambda i: (0, i, 0)),
            pl.BlockSpec((_NC, _B, _DH), lambda i: (0, i, 0)),
            pl.BlockSpec((_B, _D), lambda i: (i, 0)),
            pl.BlockSpec((_B, _D), lambda i: (i, 0)),
            pl.BlockSpec((_B, _D), lambda i: (i, 0)),
            pl.BlockSpec((_D, _D), lambda i: (0, 0)),
            pl.BlockSpec((_D, _D), lambda i: (0, 0)),
            pl.BlockSpec((1, _D), lambda i: (0, 0)),
        ],
        out_specs=[pl.BlockSpec((_B, _D), lambda i: (i, 0))] * 2,
        out_shape=[o, o],
    )(x, parts_a, parts_b, xn, ci, rs, WaT, WbT, b)


def _tc_layer2(x, parts_a, parts_b, xn, ci, WaT, WbT, b):
    def body(x_ref, pa_ref, pb_ref, xn_ref, ci_ref, wa_ref, wb_ref, b_ref,
             lo_ref, hi_ref):
        pa = pa_ref[...]
        pb = pb_ref[...]
        summed = jnp.concatenate([pa[0] + pa[1], pb[0] + pb[1]], axis=-1)
        xu = (summed + xn_ref[...]) * ci_ref[...]
        h = jnp.dot(x_ref[...], wa_ref[...], preferred_element_type=jnp.float32,
                    precision=lax.Precision.HIGHEST)
        h = h + jnp.dot(xu, wb_ref[...], preferred_element_type=jnp.float32,
                        precision=lax.Precision.HIGHEST)
        h = h + b_ref[...]
        lo_ref[...] = h[:, :_DH]
        hi_ref[...] = h[:, _DH:]

    return pl.pallas_call(
        body,
        grid=(_N // _B,),
        in_specs=[
            pl.BlockSpec((_B, _D), lambda i: (i, 0)),
            pl.BlockSpec((_NC, _B, _DH), lambda i: (0, i, 0)),
            pl.BlockSpec((_NC, _B, _DH), lambda i: (0, i, 0)),
            pl.BlockSpec((_B, _D), lambda i: (i, 0)),
            pl.BlockSpec((_B, _D), lambda i: (i, 0)),
            pl.BlockSpec((_D, _D), lambda i: (0, 0)),
            pl.BlockSpec((_D, _D), lambda i: (0, 0)),
            pl.BlockSpec((1, _D), lambda i: (0, 0)),
        ],
        out_specs=[pl.BlockSpec((_B, _DH), lambda i: (i, 0))] * 2,
        out_shape=[jax.ShapeDtypeStruct((_N, _DH), jnp.float32)] * 2,
    )(x, parts_a, parts_b, xn, ci, WaT, WbT, b)


def _tc_mlp(hs_lo, hs_hi, hr_lo, hr_hi, P1wT, p1b, p2, p2b):
    def body(sl_ref, sh_ref, rl_ref, rh_ref, w_ref, b_ref, p2_ref, p2b_ref,
             o_ref):
        z = jnp.concatenate(
            [sl_ref[...] * rl_ref[...], sh_ref[...] * rh_ref[...]], axis=-1)
        a = jnp.dot(z, w_ref[...], preferred_element_type=jnp.float32,
                    precision=lax.Precision.HIGHEST)
        a = jnp.maximum(a + b_ref[...], 0.0)
        o_ref[...] = jnp.sum(a * p2_ref[...], axis=1, keepdims=True) + p2b_ref[...]

    half = pl.BlockSpec((_B, _DH), lambda i: (i, 0))
    return pl.pallas_call(
        body,
        grid=(_P // _B,),
        in_specs=[
            half, half, half, half,
            pl.BlockSpec((_D, _D), lambda i: (0, 0)),
            pl.BlockSpec((1, _D), lambda i: (0, 0)),
            pl.BlockSpec((1, _D), lambda i: (0, 0)),
            pl.BlockSpec((1, 1), lambda i: (0, 0)),
        ],
        out_specs=pl.BlockSpec((_B, 1), lambda i: (i, 0)),
        out_shape=jax.ShapeDtypeStruct((_P, 1), jnp.float32),
    )(hs_lo, hs_hi, hr_lo, hr_hi, P1wT, p1b, p2, p2b)


def kernel(node_gids, senders, receivers, pairs, emb, W1, b1, W2, b2,
           P1w, P1b, P2w, P2b):
    f32 = jnp.float32
    x = emb

    epad_h = jnp.full((_EPAD - _E,), _DUMMY, jnp.int32)
    epad_g = jnp.zeros((_EPAD - _E,), jnp.int32)
    s_hist = jnp.concatenate([senders, epad_h]).reshape(_EROWS, _CHUNK)
    r_idx = jnp.concatenate([receivers, epad_h]).reshape(_EROWS, _CHUNK)
    s_gath = jnp.concatenate([senders, epad_g]).reshape(_EROWS, _CHUNK)
    ppad = jnp.zeros((_PPAD - _P,), jnp.int32)
    p_idx = jnp.concatenate(
        [pairs[:, 0], ppad, pairs[:, 1], ppad]).reshape(2 * _PROWS, _CHUNK)

    z64 = jnp.zeros((_STRIPE, _DH), f32)
    z16 = jnp.zeros((_STRIPE, 16), f32)
    ones16 = jnp.ones((_CHUNK, 16), f32)

    W1aT = W1[:, :_D].T
    W1bT = W1[:, _D:].T
    W2aT = W2[:, :_D].T
    W2bT = W2[:, _D:].T

    hist = _sc_hist(s_hist, r_idx, ones16, z16)
    xn1, ci, rs = _tc_prep(hist, x)
    parts1a = _sc_scatter(xn1[:, :_DH], s_gath, r_idx, z64)
    parts1b = _sc_scatter(xn1[:, _DH:], s_gath, r_idx, z64)
    h1, xn2 = _tc_layer1(x, parts1a, parts1b, xn1, ci, rs, W1aT, W1bT,
                         b1.reshape(1, _D))
    parts2a = _sc_scatter(xn2[:, :_DH], s_gath, r_idx, z64)
    parts2b = _sc_scatter(xn2[:, _DH:], s_gath, r_idx, z64)
    h2_lo, h2_hi = _tc_layer2(h1, parts2a, parts2b, xn2, ci, W2aT, W2bT,
                              b2.reshape(1, _D))
    g_lo, g_hi = _sc_pair_gather(h2_lo, h2_hi, p_idx)
    scores = _tc_mlp(g_lo[:_P], g_hi[:_P], g_lo[_PPAD:_PPAD + _P],
                     g_hi[_PPAD:_PPAD + _P], P1w.T, P1b.reshape(1, _D),
                     P2w.reshape(1, _D), P2b.reshape(1, 1))
    return scores.reshape(_P)


# speedup vs baseline: 1.3424x; 1.3424x over previous
import functools

import jax
import jax.numpy as jnp
from jax import lax
from jax.experimental import pallas as pl
from jax.experimental.pallas import tpu as pltpu
from jax.experimental.pallas import tpu_sc as plsc

_N = 10000
_D = 128
_E = 320000
_P = 100000

_NC, _NS = 2, 16
_NW = _NC * _NS

_ACC = 10240
_STRIPE = _ACC // _NS
_DUMMY = _ACC - 1

_CHUNK = 128
_EROWS = 2560
_EPT = _EROWS // _NW
_EPAD = _EROWS * _CHUNK

_PROWS = 896
_PPAD = _PROWS * _CHUNK
_PRT = 2 * _PROWS // _NW

_B = 1000
_DH = 64

_mesh = plsc.VectorSubcoreMesh(core_axis_name="c", subcore_axis_name="s")


def _sc_hist(s_idx, r_idx, ones16, z16):
    @functools.partial(
        pl.kernel,
        out_type=jax.ShapeDtypeStruct((_NC, 2, _ACC, 16), jnp.float32),
        mesh=_mesh,
        compiler_params=pltpu.CompilerParams(use_tc_tiling_on_sc=False),
        scratch_types=[
            pltpu.VMEM((_EPT, _CHUNK), jnp.int32),
            pltpu.VMEM((_EPT, _CHUNK), jnp.int32),
            pltpu.VMEM((_CHUNK, 16), jnp.float32),
            pltpu.VMEM_SHARED((_ACC, 16), jnp.float32),
            pltpu.VMEM_SHARED((_ACC, 16), jnp.float32),
        ] + [pltpu.SemaphoreType.DMA] * 8,
    )
    def hist_kernel(s_hbm, r_hbm, ones_hbm, z_hbm, out_hbm, s_v, r_v, ones_v,
                    acc_s, acc_r, *hsems):
        cid = lax.axis_index("c")
        sid = lax.axis_index("s")
        row0 = (sid * _NC + cid) * _EPT
        stripe = pl.ds(sid * _STRIPE, _STRIPE)
        pltpu.sync_copy(z_hbm, acc_s.at[stripe])
        pltpu.sync_copy(z_hbm, acc_r.at[stripe])
        pltpu.sync_copy(s_hbm.at[pl.ds(row0, _EPT)], s_v)
        pltpu.sync_copy(r_hbm.at[pl.ds(row0, _EPT)], r_v)
        pltpu.sync_copy(ones_hbm, ones_v)
        plsc.subcore_barrier()

        @pl.loop(0, _EPT, step=4)
        def _(j):
            cps = []
            for k in range(4):
                cps.append(pltpu.async_copy(
                    ones_v, acc_s.at[s_v.at[j + k]], hsems[k], add=True))
                cps.append(pltpu.async_copy(
                    ones_v, acc_r.at[r_v.at[j + k]], hsems[4 + k], add=True))
            for cp in cps:
                cp.wait()

        plsc.subcore_barrier()
        pltpu.sync_copy(acc_s.at[stripe], out_hbm.at[cid, 0, stripe])
        pltpu.sync_copy(acc_r.at[stripe], out_hbm.at[cid, 1, stripe])

    return hist_kernel(s_idx, r_idx, ones16, z16)


def _sc_scatter(src, s_idx, r_idx, z64):
    @functools.partial(
        pl.kernel,
        out_type=jax.ShapeDtypeStruct((_NC, _ACC, _DH), jnp.float32),
        mesh=_mesh,
        compiler_params=pltpu.CompilerParams(use_tc_tiling_on_sc=False),
        scratch_types=[
            pltpu.VMEM((_EPT, _CHUNK), jnp.int32),
            pltpu.VMEM((_EPT, _CHUNK), jnp.int32),
        ] + [pltpu.VMEM((_CHUNK, _DH), jnp.float32)] * 8
          + [pltpu.VMEM_SHARED((_ACC, _DH), jnp.float32)]
          + [pltpu.SemaphoreType.DMA] * 16,
    )
    def scat_kernel(x_hbm, s_hbm, r_hbm, z_hbm, out_hbm, s_v, r_v, *rest):
        bufs = rest[:8]
        acc = rest[8]
        gsems = rest[9:17]
        ssems = rest[17:25]
        cid = lax.axis_index("c")
        sid = lax.axis_index("s")
        row0 = (sid * _NC + cid) * _EPT
        stripe = pl.ds(sid * _STRIPE, _STRIPE)
        pltpu.sync_copy(z_hbm, acc.at[stripe])
        pltpu.sync_copy(s_hbm.at[pl.ds(row0, _EPT)], s_v)
        pltpu.sync_copy(r_hbm.at[pl.ds(row0, _EPT)], r_v)
        plsc.subcore_barrier()

        @pl.loop(0, _EPT, step=8)
        def _(j):
            gs = [
                pltpu.async_copy(x_hbm.at[s_v.at[j + k]], bufs[k], gsems[k])
                for k in range(8)
            ]
            ss = []
            for k in range(8):
                gs[k].wait()
                ss.append(pltpu.async_copy(
                    bufs[k], acc.at[r_v.at[j + k]], ssems[k], add=True))
            for cp in ss:
                cp.wait()

        plsc.subcore_barrier()
        pltpu.sync_copy(acc.at[stripe], out_hbm.at[cid, stripe])

    return scat_kernel(src, s_idx, r_idx, z64)


def _sc_pair_mul(h_part, p_idx, w):
    @functools.partial(
        pl.kernel,
        out_type=jax.ShapeDtypeStruct((_PPAD, w), jnp.float32),
        mesh=_mesh,
        compiler_params=pltpu.CompilerParams(use_tc_tiling_on_sc=False),
        scratch_types=[
            pltpu.VMEM((_PRT, _CHUNK), jnp.int32),
            pltpu.VMEM_SHARED((_N, w), jnp.float32),
        ] + [pltpu.VMEM((_CHUNK, w), jnp.float32)] * 12
          + [pltpu.SemaphoreType.DMA] * 12,
    )
    def pm_kernel(h_hbm, i_hbm, out_hbm, i_v, stage, *rest):
        abufs = rest[0:4]
        bbufs = rest[4:8]
        zbufs = rest[8:12]
        gsems = rest[12:20]
        wsems = rest[20:24]
        cid = lax.axis_index("c")
        sid = lax.axis_index("s")
        wid = sid * _NC + cid
        nck = _PROWS // _NW
        @pl.when(sid < _NS - 1)
        def _():
            pltpu.sync_copy(h_hbm.at[pl.ds(sid * 640, 640)],
                            stage.at[pl.ds(sid * 640, 640)])

        @pl.when(sid == _NS - 1)
        def _():
            pltpu.sync_copy(h_hbm.at[pl.ds((_NS - 1) * 640, _N - (_NS - 1) * 640)],
                            stage.at[pl.ds((_NS - 1) * 640, _N - (_NS - 1) * 640)])

        pltpu.sync_copy(i_hbm.at[pl.ds(wid * nck, nck)], i_v.at[pl.ds(0, nck)])
        pltpu.sync_copy(i_hbm.at[pl.ds(_PROWS + wid * nck, nck)],
                        i_v.at[pl.ds(nck, nck)])
        plsc.subcore_barrier()

        @pl.loop(0, nck, step=4)
        def _(j):
            gs = []
            for k in range(4):
                gs.append(pltpu.async_copy(
                    stage.at[i_v.at[j + k]], abufs[k], gsems[k]))
                gs.append(pltpu.async_copy(
                    stage.at[i_v.at[nck + j + k]], bbufs[k], gsems[4 + k]))
            ws = []
            for k in range(4):
                gs[2 * k].wait()
                gs[2 * k + 1].wait()

                @pl.loop(0, _CHUNK)
                def _(i):
                    for g in range(w // 16):
                        sl = pl.ds(g * 16, 16)
                        zbufs[k][i, sl] = abufs[k][i, sl] * bbufs[k][i, sl]

                ws.append(pltpu.async_copy(
                    zbufs[k],
                    out_hbm.at[pl.ds((wid * nck + j + k) * _CHUNK, _CHUNK)],
                    wsems[k]))
            for cp in ws:
                cp.wait()

    return pm_kernel(h_part, p_idx)


def _sc_pair_gather(h_lo, h_hi, p_idx):
    o = jax.ShapeDtypeStruct((2 * _PPAD, _DH), jnp.float32)

    @functools.partial(
        pl.kernel,
        out_type=[o, o],
        mesh=_mesh,
        compiler_params=pltpu.CompilerParams(use_tc_tiling_on_sc=False),
        scratch_types=[
            pltpu.VMEM((_PRT, _CHUNK), jnp.int32),
        ] + [pltpu.VMEM((_CHUNK, _DH), jnp.float32)] * 8
          + [pltpu.SemaphoreType.DMA] * 16,
    )
    def pg_kernel(lo_hbm, hi_hbm, i_hbm, out_lo, out_hi, i_v, *rest):
        bufs = rest[:8]
        gsems = rest[8:16]
        wsems = rest[16:24]
        cid = lax.axis_index("c")
        sid = lax.axis_index("s")
        row0 = (sid * _NC + cid) * _PRT
        pltpu.sync_copy(i_hbm.at[pl.ds(row0, _PRT)], i_v)
        srcs = (lo_hbm, hi_hbm)
        outs = (out_lo, out_hi)

        @pl.loop(0, _PRT, step=4)
        def _(j):
            gs = [
                pltpu.async_copy(srcs[k % 2].at[i_v.at[j + k // 2]],
                                 bufs[k], gsems[k])
                for k in range(8)
            ]
            ws = []
            for k in range(8):
                gs[k].wait()
                ws.append(pltpu.async_copy(
                    bufs[k],
                    outs[k % 2].at[pl.ds((row0 + j + k // 2) * _CHUNK, _CHUNK)],
                    wsems[k]))
            for cp in ws:
                cp.wait()

    return pg_kernel(h_lo, h_hi, p_idx)


def _tc_prep(hist, emb):
    def body(hs_ref, hr_ref, emb_ref, xn_ref, ci_ref, rs_ref):
        hsb = hs_ref[...]
        hrb = hr_ref[...]
        deg = hsb[0, 0, :, 0:1] + hsb[1, 0, :, 0:1] + 1.0
        cnt = hrb[0, 0, :, 0:1] + hrb[1, 0, :, 0:1] + 1.0
        rs = jnp.broadcast_to(lax.rsqrt(deg), (_B, _D))
        t = jnp.broadcast_to(lax.rsqrt(cnt), (_B, _D))
        xn_ref[...] = emb_ref[...] * rs
        ci_ref[...] = t * t * t
        rs_ref[...] = rs

    o = jax.ShapeDtypeStruct((_N, _D), jnp.float32)
    return pl.pallas_call(
        body,
        grid=(_N // _B,),
        in_specs=[
            pl.BlockSpec((_NC, 1, _B, 16), lambda i: (0, 0, i, 0)),
            pl.BlockSpec((_NC, 1, _B, 16), lambda i: (0, 1, i, 0)),
            pl.BlockSpec((_B, _D), lambda i: (i, 0)),
        ],
        out_specs=[pl.BlockSpec((_B, _D), lambda i: (i, 0))] * 3,
        out_shape=[o, o, o],
    )(hist, hist, emb)


def _tc_layer1(x, parts_a, parts_b, xn, ci, rs, WaT, WbT, b):
    def body(x_ref, pa_ref, pb_ref, xn_ref, ci_ref, rs_ref, wa_ref, wb_ref,
             b_ref, h_ref, xn2_ref):
        pa = pa_ref[...]
        pb = pb_ref[...]
        summed = jnp.concatenate([pa[0] + pa[1], pb[0] + pb[1]], axis=-1)
        xu = (summed + xn_ref[...]) * ci_ref[...]
        h = jnp.dot(x_ref[...], wa_ref[...], preferred_element_type=jnp.float32,
                    precision=lax.Precision.HIGHEST)
        h = h + jnp.dot(xu, wb_ref[...], preferred_element_type=jnp.float32,
                        precision=lax.Precision.HIGHEST)
        h = jnp.maximum(h + b_ref[...], 0.0)
        h_ref[...] = h
        xn2_ref[...] = h * rs_ref[...]

    o = jax.ShapeDtypeStruct((_N, _D), jnp.float32)
    return pl.pallas_call(
        body,
        grid=(_N // _B,),
        in_specs=[
            pl.BlockSpec((_B, _D), lambda i: (i, 0)),
            pl.BlockSpec((_NC, _B, _DH), lambda i: (0, i, 0)),
            pl.BlockSpec((_NC, _B, _DH), lambda i: (0, i, 0)),
            pl.BlockSpec((_B, _D), lambda i: (i, 0)),
            pl.BlockSpec((_B, _D), lambda i: (i, 0)),
            pl.BlockSpec((_B, _D), lambda i: (i, 0)),
            pl.BlockSpec((_D, _D), lambda i: (0, 0)),
            pl.BlockSpec((_D, _D), lambda i: (0, 0)),
            pl.BlockSpec((1, _D), lambda i: (0, 0)),
        ],
        out_specs=[pl.BlockSpec((_B, _D), lambda i: (i, 0))] * 2,
        out_shape=[o, o],
    )(x, parts_a, parts_b, xn, ci, rs, WaT, WbT, b)


def _tc_layer2(x, parts_a, parts_b, xn, ci, WaT, WbT, b):
    def body(x_ref, pa_ref, pb_ref, xn_ref, ci_ref, wa_ref, wb_ref, b_ref,
             lo_ref, hi_ref):
        pa = pa_ref[...]
        pb = pb_ref[...]
        summed = jnp.concatenate([pa[0] + pa[1], pb[0] + pb[1]], axis=-1)
        xu = (summed + xn_ref[...]) * ci_ref[...]
        h = jnp.dot(x_ref[...], wa_ref[...], preferred_element_type=jnp.float32,
                    precision=lax.Precision.HIGHEST)
        h = h + jnp.dot(xu, wb_ref[...], preferred_element_type=jnp.float32,
                        precision=lax.Precision.HIGHEST)
        h = h + b_ref[...]
        lo_ref[...] = h[:, :_DH]
        hi_ref[...] = h[:, _DH:]

    return pl.pallas_call(
        body,
        grid=(_N // _B,),
        in_specs=[
            pl.BlockSpec((_B, _D), lambda i: (i, 0)),
            pl.BlockSpec((_NC, _B, _DH), lambda i: (0, i, 0)),
            pl.BlockSpec((_NC, _B, _DH), lambda i: (0, i, 0)),
            pl.BlockSpec((_B, _D), lambda i: (i, 0)),
            pl.BlockSpec((_B, _D), lambda i: (i, 0)),
            pl.BlockSpec((_D, _D), lambda i: (0, 0)),
            pl.BlockSpec((_D, _D), lambda i: (0, 0)),
            pl.BlockSpec((1, _D), lambda i: (0, 0)),
        ],
        out_specs=[pl.BlockSpec((_B, _DH), lambda i: (i, 0))] * 2,
        out_shape=[jax.ShapeDtypeStruct((_N, _DH), jnp.float32)] * 2,
    )(x, parts_a, parts_b, xn, ci, WaT, WbT, b)


def _tc_mlp(z0, z1, z2, z3, P1wT, p1b, p2, p2b):
    def body(z0_ref, z1_ref, z2_ref, z3_ref, w_ref, b_ref, p2_ref, p2b_ref,
             o_ref):
        z = jnp.concatenate(
            [z0_ref[...], z1_ref[...], z2_ref[...], z3_ref[...]], axis=-1)
        a = jnp.dot(z, w_ref[...], preferred_element_type=jnp.float32,
                    precision=lax.Precision.HIGHEST)
        a = jnp.maximum(a + b_ref[...], 0.0)
        o_ref[...] = jnp.sum(a * p2_ref[...], axis=1, keepdims=True) + p2b_ref[...]

    quarter = pl.BlockSpec((_B, _DH // 2), lambda i: (i, 0))
    return pl.pallas_call(
        body,
        grid=(_P // _B,),
        in_specs=[
            quarter, quarter, quarter, quarter,
            pl.BlockSpec((_D, _D), lambda i: (0, 0)),
            pl.BlockSpec((1, _D), lambda i: (0, 0)),
            pl.BlockSpec((1, _D), lambda i: (0, 0)),
            pl.BlockSpec((1, 1), lambda i: (0, 0)),
        ],
        out_specs=pl.BlockSpec((_B, 1), lambda i: (i, 0)),
        out_shape=jax.ShapeDtypeStruct((_P, 1), jnp.float32),
    )(z0, z1, z2, z3, P1wT, p1b, p2, p2b)


def kernel(node_gids, senders, receivers, pairs, emb, W1, b1, W2, b2,
           P1w, P1b, P2w, P2b):
    f32 = jnp.float32
    x = emb

    epad_h = jnp.full((_EPAD - _E,), _DUMMY, jnp.int32)
    epad_g = jnp.zeros((_EPAD - _E,), jnp.int32)
    s_hist = jnp.concatenate([senders, epad_h]).reshape(_EROWS, _CHUNK)
    r_idx = jnp.concatenate([receivers, epad_h]).reshape(_EROWS, _CHUNK)
    s_gath = jnp.concatenate([senders, epad_g]).reshape(_EROWS, _CHUNK)
    ppad = jnp.zeros((_PPAD - _P,), jnp.int32)
    p_idx = jnp.concatenate(
        [pairs[:, 0], ppad, pairs[:, 1], ppad]).reshape(2 * _PROWS, _CHUNK)

    z64 = jnp.zeros((_STRIPE, _DH), f32)
    z16 = jnp.zeros((_STRIPE, 16), f32)
    ones16 = jnp.ones((_CHUNK, 16), f32)

    W1aT = W1[:, :_D].T
    W1bT = W1[:, _D:].T
    W2aT = W2[:, :_D].T
    W2bT = W2[:, _D:].T

    hist = _sc_hist(s_hist, r_idx, ones16, z16)
    xn1, ci, rs = _tc_prep(hist, x)
    parts1a = _sc_scatter(xn1[:, :_DH], s_gath, r_idx, z64)
    parts1b = _sc_scatter(xn1[:, _DH:], s_gath, r_idx, z64)
    h1, xn2 = _tc_layer1(x, parts1a, parts1b, xn1, ci, rs, W1aT, W1bT,
                         b1.reshape(1, _D))
    parts2a = _sc_scatter(xn2[:, :_DH], s_gath, r_idx, z64)
    parts2b = _sc_scatter(xn2[:, _DH:], s_gath, r_idx, z64)
    h2_lo, h2_hi = _tc_layer2(h1, parts2a, parts2b, xn2, ci, W2aT, W2bT,
                              b2.reshape(1, _D))
    q = _DH // 2
    zq = [_sc_pair_mul(part, p_idx, q)[:_P]
          for part in (h2_lo[:, :q], h2_lo[:, q:], h2_hi[:, :q], h2_hi[:, q:])]
    scores = _tc_mlp(*zq, P1w.T, P1b.reshape(1, _D),
                     P2w.reshape(1, _D), P2b.reshape(1, 1))
    return scores.reshape(_P)
